# Initial kernel scaffold; baseline (speedup 1.0000x reference)
import jax
import jax.numpy as jnp
from jax.experimental import pallas as pl


def kernel(x, edge_index, edge_weight, W1, b1, W2, b2, W3, b3, g1, be1, g2, be2, g3, be3, linW, linb):
    raise NotImplementedError("write your pallas kernel here")


# speedup vs baseline: 8.9149x; 8.9149x over previous
import functools
import jax
import jax.numpy as jnp
from jax import lax
from jax.experimental import pallas as pl
from jax.experimental.pallas import tpu as pltpu
from jax.experimental.pallas import tpu_sc as plsc

F = 128
EPSBN = 1e-5
NWORKERS = 32
CH = 128
TPB = 640


def _mesh():
    return plsc.VectorSubcoreMesh(core_axis_name="c", subcore_axis_name="s")


def _make_deg(E, N, NPAD):
    epw = E // NWORKERS
    nfull = epw // CH
    tail = epw - nfull * CH

    @functools.partial(
        pl.kernel,
        out_type=jax.ShapeDtypeStruct((2, NPAD), jnp.float32),
        mesh=_mesh(),
        compiler_params=pltpu.CompilerParams(needs_layout_passes=False),
        scratch_types=[
            pltpu.VMEM((CH,), jnp.int32),
            pltpu.VMEM((CH,), jnp.float32),
            pltpu.VMEM((tail,), jnp.int32) if tail else None,
            pltpu.VMEM((tail,), jnp.float32) if tail else None,
            pltpu.VMEM_SHARED((NPAD,), jnp.float32),
        ],
    )
    def deg_kernel(col_hbm, ew_hbm, z_hbm, out_hbm, col_v, ew_v, col_t, ew_t,
                   acc_sh):
        cid = lax.axis_index("c")
        sid = lax.axis_index("s")
        wid = sid * 2 + cid
        base = wid * epw
        pltpu.sync_copy(z_hbm, acc_sh.at[pl.ds(sid * TPB, TPB)])
        plsc.subcore_barrier()

        def body(j, _):
            off = base + j * CH
            pltpu.sync_copy(col_hbm.at[pl.ds(off, CH)], col_v)
            pltpu.sync_copy(ew_hbm.at[pl.ds(off, CH)], ew_v)
            pltpu.sync_copy(ew_v, acc_sh.at[col_v], add=True)
            return 0

        lax.fori_loop(0, nfull, body, 0)
        if tail:
            off = base + nfull * CH
            pltpu.sync_copy(col_hbm.at[pl.ds(off, tail)], col_t)
            pltpu.sync_copy(ew_hbm.at[pl.ds(off, tail)], ew_t)
            pltpu.sync_copy(ew_t, acc_sh.at[col_t], add=True)
        plsc.subcore_barrier()
        pltpu.sync_copy(acc_sh.at[pl.ds(sid * TPB, TPB)],
                        out_hbm.at[cid, pl.ds(sid * TPB, TPB)])

    return deg_kernel


def _make_prop(E, N, NPAD):
    epw = E // NWORKERS
    nfull = epw // CH
    tail = epw - nfull * CH

    @functools.partial(
        pl.kernel,
        out_type=jax.ShapeDtypeStruct((2, NPAD, F), jnp.float32),
        mesh=_mesh(),
        compiler_params=pltpu.CompilerParams(needs_layout_passes=False),
        scratch_types=[
            pltpu.VMEM((CH,), jnp.int32),
            pltpu.VMEM((CH,), jnp.int32),
            pltpu.VMEM((CH,), jnp.float32),
            pltpu.VMEM((CH, F), jnp.float32),
            pltpu.VMEM((tail,), jnp.int32) if tail else None,
            pltpu.VMEM((tail,), jnp.int32) if tail else None,
            pltpu.VMEM((tail,), jnp.float32) if tail else None,
            pltpu.VMEM((tail, F), jnp.float32) if tail else None,
            pltpu.VMEM_SHARED((NPAD, F), jnp.float32),
            pltpu.SemaphoreType.DMA,
        ],
    )
    def prop_kernel(hp_hbm, row_hbm, col_hbm, ew_hbm, z_hbm, out_hbm,
                    row_v, col_v, ew_v, rows_v, row_t, col_t, ew_t, rows_t,
                    acc_sh, sem):
        cid = lax.axis_index("c")
        sid = lax.axis_index("s")
        wid = sid * 2 + cid
        base = wid * epw
        pltpu.sync_copy(z_hbm, acc_sh.at[pl.ds(sid * TPB, TPB)])
        plsc.subcore_barrier()

        def scale(n, idx_ref, ew_ref, buf_ref):
            def sbody(e, _):
                ewb = plsc.load_gather(
                    ew_ref, [jnp.full((16,), e, dtype=jnp.int32)])
                for jj in range(F // 16):
                    sl = pl.ds(jj * 16, 16)
                    buf_ref[e, sl] = buf_ref[e, sl] * ewb
                return 0

            lax.fori_loop(0, n, sbody, 0)

        def chunk(j, _):
            off = base + j * CH
            pltpu.sync_copy(row_hbm.at[pl.ds(off, CH)], row_v)
            pltpu.sync_copy(col_hbm.at[pl.ds(off, CH)], col_v)
            pltpu.sync_copy(ew_hbm.at[pl.ds(off, CH)], ew_v)
            pltpu.async_copy(hp_hbm.at[row_v], rows_v, sem).wait()
            scale(CH, row_v, ew_v, rows_v)
            pltpu.sync_copy(rows_v, acc_sh.at[col_v], add=True)
            return 0

        lax.fori_loop(0, nfull, chunk, 0)
        if tail:
            off = base + nfull * CH
            pltpu.sync_copy(row_hbm.at[pl.ds(off, tail)], row_t)
            pltpu.sync_copy(col_hbm.at[pl.ds(off, tail)], col_t)
            pltpu.sync_copy(ew_hbm.at[pl.ds(off, tail)], ew_t)
            pltpu.async_copy(hp_hbm.at[row_t], rows_t, sem).wait()
            scale(tail, row_t, ew_t, rows_t)
            pltpu.sync_copy(rows_t, acc_sh.at[col_t], add=True)
        plsc.subcore_barrier()
        pltpu.sync_copy(acc_sh.at[pl.ds(sid * TPB, TPB)],
                        out_hbm.at[cid, pl.ds(sid * TPB, TPB)])

    return prop_kernel


RB = 2000


def _mm(a, b):
    return jax.lax.dot_general(a, b, (((1,), (0,)), ((), ())),
                               precision=jax.lax.Precision.HIGHEST,
                               preferred_element_type=jnp.float32)


def _tc0_body(N, degp_ref, dinv_ref):
    deg = degp_ref[0, :] + degp_ref[1, :] + 1.0
    dinv_ref[...] = jax.lax.rsqrt(deg)[:, None]


def _tc1_body(x_ref, w_ref, dinv_ref, hp_ref):
    hp_ref[...] = _mm(x_ref[...], w_ref[...]) * dinv_ref[...]


def _tca_body(relu, s_ref, hp_ref, dinv_ref, b_ref, r_ref, st_ref):
    i = pl.program_id(0)
    t = (s_ref[0] + s_ref[1] + hp_ref[...]) * dinv_ref[...] + b_ref[...]
    if relu:
        t = jnp.maximum(t, 0.0)
    r_ref[...] = t

    @pl.when(i == 0)
    def _():
        st_ref[...] = jnp.zeros_like(st_ref)

    st_ref[0, :] += jnp.sum(t, axis=0)
    st_ref[1, :] += jnp.sum(t * t, axis=0)


def _stats(N, st_ref):
    mean = st_ref[0, :] * (1.0 / N)
    var = st_ref[1, :] * (1.0 / N) - mean * mean
    return mean[None, :], jax.lax.rsqrt(var + EPSBN)[None, :]


def _tcb_body(N, r_ref, st_ref, g_ref, be_ref, wn_ref, dinv_ref,
              x_ref, hn_ref):
    mean, rstd = _stats(N, st_ref)
    xn = (r_ref[...] - mean) * rstd * g_ref[...] + be_ref[...]
    x_ref[...] = xn
    hn_ref[...] = _mm(xn, wn_ref[...]) * dinv_ref[...]


def _tc3_body(N, r_ref, st_ref, g_ref, be_ref, x1_ref, x2_ref,
              linw_ref, linb_ref, logits_ref, probs_ref, embed_ref):
    mean, rstd = _stats(N, st_ref)
    x3 = (r_ref[...] - mean) * rstd * g_ref[...] + be_ref[...]
    x1 = x1_ref[...]
    x2 = x2_ref[...]
    embed_ref[:, 0:F] = x1
    embed_ref[:, F:2 * F] = x2
    embed_ref[:, 2 * F:3 * F] = x3
    logits = (_mm(x1, linw_ref[0:F, :]) + _mm(x2, linw_ref[F:2 * F, :])
              + _mm(x3, linw_ref[2 * F:3 * F, :]) + linb_ref[...])
    logits_ref[...] = logits
    m = jnp.max(logits, axis=1, keepdims=True)
    e = jnp.exp(logits - m)
    probs_ref[...] = e / jnp.sum(e, axis=1, keepdims=True)


def _row_spec(width):
    return pl.BlockSpec((RB, width), lambda i: (i, 0))


def _full_spec():
    return pl.BlockSpec(lambda i: tuple())


def kernel(x, edge_index, edge_weight, W1, b1, W2, b2, W3, b3,
           g1, be1, g2, be2, g3, be3, linW, linb):
    N = x.shape[0]
    E = edge_weight.shape[0]
    NPAD = 16 * TPB
    G = N // RB
    nclass = linW.shape[1]
    f32 = jnp.float32
    row = edge_index[0]
    col = edge_index[1]
    z1 = jnp.zeros((TPB,), f32)
    z2 = jnp.zeros((TPB, F), f32)

    vec128 = pl.BlockSpec((F,), lambda i: (0,))

    degp = _make_deg(E, N, NPAD)(col, edge_weight, z1)

    dinv = pl.pallas_call(
        functools.partial(_tc0_body, N),
        out_shape=jax.ShapeDtypeStruct((N, 1), f32),
    )(degp[:, :N])

    tc1 = pl.pallas_call(
        _tc1_body,
        grid=(G,),
        in_specs=[_row_spec(F),
                  pl.BlockSpec((F, F), lambda i: (0, 0)),
                  _row_spec(1)],
        out_specs=_row_spec(F),
        out_shape=jax.ShapeDtypeStruct((N, F), f32),
    )
    hp = tc1(x, W1, dinv)

    prop = _make_prop(E, N, NPAD)

    def tca(relu, s, hp_, b):
        return pl.pallas_call(
            functools.partial(_tca_body, relu),
            grid=(G,),
            in_specs=[pl.BlockSpec((2, RB, F), lambda i: (0, i, 0)),
                      _row_spec(F), _row_spec(1), vec128],
            out_specs=(_row_spec(F),
                       pl.BlockSpec((2, F), lambda i: (0, 0))),
            out_shape=(jax.ShapeDtypeStruct((N, F), f32),
                       jax.ShapeDtypeStruct((2, F), f32)),
        )(s, hp_, dinv, b)

    def tcb(r, st, g, be, wn):
        return pl.pallas_call(
            functools.partial(_tcb_body, N),
            grid=(G,),
            in_specs=[_row_spec(F),
                      pl.BlockSpec((2, F), lambda i: (0, 0)),
                      vec128, vec128,
                      pl.BlockSpec((F, F), lambda i: (0, 0)),
                      _row_spec(1)],
            out_specs=(_row_spec(F), _row_spec(F)),
            out_shape=(jax.ShapeDtypeStruct((N, F), f32),
                       jax.ShapeDtypeStruct((N, F), f32)),
        )(r, st, g, be, wn, dinv)

    s1 = prop(hp, row, col, edge_weight, z2)
    r1, st1 = tca(True, s1, hp, b1)
    x1, hp2 = tcb(r1, st1, g1, be1, W2)
    s2 = prop(hp2, row, col, edge_weight, z2)
    r2, st2 = tca(True, s2, hp2, b2)
    x2, hp3 = tcb(r2, st2, g2, be2, W3)
    s3 = prop(hp3, row, col, edge_weight, z2)
    r3, st3 = tca(False, s3, hp3, b3)

    logits, probs, embed = pl.pallas_call(
        functools.partial(_tc3_body, N),
        grid=(G,),
        in_specs=[_row_spec(F),
                  pl.BlockSpec((2, F), lambda i: (0, 0)),
                  vec128, vec128,
                  _row_spec(F), _row_spec(F),
                  pl.BlockSpec((3 * F, nclass), lambda i: (0, 0)),
                  pl.BlockSpec((nclass,), lambda i: (0,))],
        out_specs=(_row_spec(nclass), _row_spec(nclass), _row_spec(3 * F)),
        out_shape=(jax.ShapeDtypeStruct((N, nclass), f32),
                   jax.ShapeDtypeStruct((N, nclass), f32),
                   jax.ShapeDtypeStruct((N, 3 * F), f32)),
    )(r3, st3, g3, be3, x1, x2, linW, linb)
    return (logits, probs, embed)


# speedup vs baseline: 13.7083x; 1.5377x over previous
import functools
import jax
import jax.numpy as jnp
from jax import lax
from jax.experimental import pallas as pl
from jax.experimental.pallas import tpu as pltpu
from jax.experimental.pallas import tpu_sc as plsc

F = 128
EPSBN = 1e-5
NWORKERS = 32
CH = 128
TPB = 640


def _mesh():
    return plsc.VectorSubcoreMesh(core_axis_name="c", subcore_axis_name="s")


def _make_deg(E, N, NPAD):
    epw = E // NWORKERS
    nfull = epw // CH
    tail = epw - nfull * CH

    @functools.partial(
        pl.kernel,
        out_type=jax.ShapeDtypeStruct((2, NPAD), jnp.float32),
        mesh=_mesh(),
        compiler_params=pltpu.CompilerParams(needs_layout_passes=False),
        scratch_types=[
            pltpu.VMEM((CH,), jnp.int32),
            pltpu.VMEM((CH,), jnp.float32),
            pltpu.VMEM((tail,), jnp.int32) if tail else None,
            pltpu.VMEM((tail,), jnp.float32) if tail else None,
            pltpu.VMEM_SHARED((NPAD,), jnp.float32),
        ],
    )
    def deg_kernel(col_hbm, ew_hbm, z_hbm, out_hbm, col_v, ew_v, col_t, ew_t,
                   acc_sh):
        cid = lax.axis_index("c")
        sid = lax.axis_index("s")
        wid = sid * 2 + cid
        base = wid * epw
        pltpu.sync_copy(z_hbm, acc_sh.at[pl.ds(sid * TPB, TPB)])
        plsc.subcore_barrier()

        def body(j, _):
            off = base + j * CH
            pltpu.sync_copy(col_hbm.at[pl.ds(off, CH)], col_v)
            pltpu.sync_copy(ew_hbm.at[pl.ds(off, CH)], ew_v)
            pltpu.sync_copy(ew_v, acc_sh.at[col_v], add=True)
            return 0

        lax.fori_loop(0, nfull, body, 0)
        if tail:
            off = base + nfull * CH
            pltpu.sync_copy(col_hbm.at[pl.ds(off, tail)], col_t)
            pltpu.sync_copy(ew_hbm.at[pl.ds(off, tail)], ew_t)
            pltpu.sync_copy(ew_t, acc_sh.at[col_t], add=True)
        plsc.subcore_barrier()
        pltpu.sync_copy(acc_sh.at[pl.ds(sid * TPB, TPB)],
                        out_hbm.at[cid, pl.ds(sid * TPB, TPB)])

    return deg_kernel


def _make_prop(E, N, NPAD):
    epw = E // NWORKERS
    nfull = epw // CH
    tail = epw - nfull * CH
    nhalf = nfull // 2
    assert nfull % 2 == 0

    @functools.partial(
        pl.kernel,
        out_type=jax.ShapeDtypeStruct((2, NPAD, F), jnp.float32),
        mesh=_mesh(),
        compiler_params=pltpu.CompilerParams(needs_layout_passes=False),
        scratch_types=[
            pltpu.VMEM((2, CH), jnp.int32),
            pltpu.VMEM((2, CH), jnp.int32),
            pltpu.VMEM((2, CH), jnp.int32),
            pltpu.VMEM((2, CH), jnp.float32),
            pltpu.VMEM((2, CH, F), jnp.float32),
            pltpu.VMEM((tail,), jnp.int32),
            pltpu.VMEM((tail,), jnp.int32),
            pltpu.VMEM((tail,), jnp.float32),
            pltpu.VMEM((tail, F), jnp.float32),
            pltpu.VMEM_SHARED((NPAD, F), jnp.float32),
            pltpu.SemaphoreType.DMA((2,)),
            pltpu.SemaphoreType.DMA((2,)),
            pltpu.SemaphoreType.DMA((2,)),
            pltpu.SemaphoreType.DMA,
        ],
    )
    def prop_kernel(hp_hbm, row_hbm, col_hbm, ew_hbm, z_hbm, out_hbm,
                    rowi_v, coli_v, colsc_v, ew_v, rows_v,
                    row_t, col_t, ew_t, rows_t,
                    acc_sh, sem_i, sem_g, sem_s, sem_t):
        cid = lax.axis_index("c")
        sid = lax.axis_index("s")
        wid = sid * 2 + cid
        base = wid * epw
        pltpu.sync_copy(z_hbm, acc_sh.at[pl.ds(sid * TPB, TPB)])
        plsc.subcore_barrier()

        def issue_idx(j, b):
            off = base + j * CH
            pltpu.async_copy(row_hbm.at[pl.ds(off, CH)], rowi_v.at[b],
                             sem_i.at[b])
            pltpu.async_copy(col_hbm.at[pl.ds(off, CH)], coli_v.at[b],
                             sem_i.at[b])
            pltpu.async_copy(ew_hbm.at[pl.ds(off, CH)], ew_v.at[b],
                             sem_i.at[b])

        def wait_idx(b):
            pltpu.make_async_copy(row_hbm.at[pl.ds(0, CH)], rowi_v.at[b],
                                  sem_i.at[b]).wait()
            pltpu.make_async_copy(col_hbm.at[pl.ds(0, CH)], coli_v.at[b],
                                  sem_i.at[b]).wait()
            pltpu.make_async_copy(ew_hbm.at[pl.ds(0, CH)], ew_v.at[b],
                                  sem_i.at[b]).wait()

        def issue_gather(b):
            pltpu.async_copy(hp_hbm.at[rowi_v.at[b]], rows_v.at[b],
                             sem_g.at[b])

        def wait_gather(b):
            pltpu.make_async_copy(hp_hbm.at[rowi_v.at[b]], rows_v.at[b],
                                  sem_g.at[b]).wait()

        def issue_scatter(b):
            for jj in range(CH // 16):
                sl = pl.ds(jj * 16, 16)
                colsc_v[b, sl] = coli_v[b, sl]
            pltpu.async_copy(rows_v.at[b], acc_sh.at[colsc_v.at[b]],
                             sem_s.at[b], add=True)

        def wait_scatter(b):
            pltpu.make_async_copy(rows_v.at[b], acc_sh.at[colsc_v.at[b]],
                                  sem_s.at[b]).wait()

        def scale(n, b, ew_ref, buf_ref):
            def sbody(e, _):
                ewb = plsc.load_gather(
                    ew_ref, [jnp.full((16,), e, dtype=jnp.int32)])
                for jj in range(F // 16):
                    sl = pl.ds(jj * 16, 16)
                    buf_ref[b, e, sl] = buf_ref[b, e, sl] * ewb
                return 0

            lax.fori_loop(0, n, sbody, 0, unroll=2)

        issue_idx(0, 0)
        issue_idx(1, 1)
        wait_idx(0)
        issue_gather(0)

        def body(t, _):
            wait_gather(0)
            scale(CH, 0, ew_v.at[0], rows_v)
            issue_scatter(0)

            @pl.when(t < nhalf - 1)
            def _():
                issue_idx(2 * t + 2, 0)

            wait_idx(1)

            @pl.when(t > 0)
            def _():
                wait_scatter(1)

            issue_gather(1)
            wait_gather(1)
            scale(CH, 1, ew_v.at[1], rows_v)
            issue_scatter(1)

            @pl.when(t < nhalf - 1)
            def _():
                issue_idx(2 * t + 3, 1)
                wait_idx(0)
                wait_scatter(0)
                issue_gather(0)

            return 0

        lax.fori_loop(0, nhalf, body, 0)
        wait_scatter(0)
        wait_scatter(1)
        if tail:
            off = base + nfull * CH
            pltpu.sync_copy(row_hbm.at[pl.ds(off, tail)], row_t)
            pltpu.sync_copy(col_hbm.at[pl.ds(off, tail)], col_t)
            pltpu.sync_copy(ew_hbm.at[pl.ds(off, tail)], ew_t)
            pltpu.async_copy(hp_hbm.at[row_t], rows_t, sem_t).wait()

            def sbody(e, _):
                ewb = plsc.load_gather(
                    ew_t, [jnp.full((16,), e, dtype=jnp.int32)])
                for jj in range(F // 16):
                    sl = pl.ds(jj * 16, 16)
                    rows_t[e, sl] = rows_t[e, sl] * ewb
                return 0

            lax.fori_loop(0, tail, sbody, 0)
            pltpu.sync_copy(rows_t, acc_sh.at[col_t], add=True)
        plsc.subcore_barrier()
        pltpu.sync_copy(acc_sh.at[pl.ds(sid * TPB, TPB)],
                        out_hbm.at[cid, pl.ds(sid * TPB, TPB)])

    return prop_kernel


RB = 2000


def _mm(a, b):
    return jax.lax.dot_general(a, b, (((1,), (0,)), ((), ())),
                               precision=jax.lax.Precision.HIGHEST,
                               preferred_element_type=jnp.float32)


def _tc0_body(N, degp_ref, dinv_ref):
    deg = degp_ref[0, :] + degp_ref[1, :] + 1.0
    dinv_ref[...] = jax.lax.rsqrt(deg)[:, None]


def _tc1_body(x_ref, w_ref, dinv_ref, hp_ref):
    hp_ref[...] = _mm(x_ref[...], w_ref[...]) * dinv_ref[...]


def _tca_body(relu, s_ref, hp_ref, dinv_ref, b_ref, r_ref, st_ref):
    i = pl.program_id(0)
    t = (s_ref[0] + s_ref[1] + hp_ref[...]) * dinv_ref[...] + b_ref[...]
    if relu:
        t = jnp.maximum(t, 0.0)
    r_ref[...] = t

    @pl.when(i == 0)
    def _():
        st_ref[...] = jnp.zeros_like(st_ref)

    st_ref[0, :] += jnp.sum(t, axis=0)
    st_ref[1, :] += jnp.sum(t * t, axis=0)


def _stats(N, st_ref):
    mean = st_ref[0, :] * (1.0 / N)
    var = st_ref[1, :] * (1.0 / N) - mean * mean
    return mean[None, :], jax.lax.rsqrt(var + EPSBN)[None, :]


def _tcb_body(N, r_ref, st_ref, g_ref, be_ref, wn_ref, dinv_ref,
              x_ref, hn_ref):
    mean, rstd = _stats(N, st_ref)
    xn = (r_ref[...] - mean) * rstd * g_ref[...] + be_ref[...]
    x_ref[...] = xn
    hn_ref[...] = _mm(xn, wn_ref[...]) * dinv_ref[...]


def _tc3_body(N, r_ref, st_ref, g_ref, be_ref, x1_ref, x2_ref,
              linw_ref, linb_ref, logits_ref, probs_ref, embed_ref):
    mean, rstd = _stats(N, st_ref)
    x3 = (r_ref[...] - mean) * rstd * g_ref[...] + be_ref[...]
    x1 = x1_ref[...]
    x2 = x2_ref[...]
    embed_ref[:, 0:F] = x1
    embed_ref[:, F:2 * F] = x2
    embed_ref[:, 2 * F:3 * F] = x3
    logits = (_mm(x1, linw_ref[0:F, :]) + _mm(x2, linw_ref[F:2 * F, :])
              + _mm(x3, linw_ref[2 * F:3 * F, :]) + linb_ref[...])
    logits_ref[...] = logits
    m = jnp.max(logits, axis=1, keepdims=True)
    e = jnp.exp(logits - m)
    probs_ref[...] = e / jnp.sum(e, axis=1, keepdims=True)


def _row_spec(width):
    return pl.BlockSpec((RB, width), lambda i: (i, 0))


def _full_spec():
    return pl.BlockSpec(lambda i: tuple())


def kernel(x, edge_index, edge_weight, W1, b1, W2, b2, W3, b3,
           g1, be1, g2, be2, g3, be3, linW, linb):
    N = x.shape[0]
    E = edge_weight.shape[0]
    NPAD = 16 * TPB
    G = N // RB
    nclass = linW.shape[1]
    f32 = jnp.float32
    row = edge_index[0]
    col = edge_index[1]
    z1 = jnp.zeros((TPB,), f32)
    z2 = jnp.zeros((TPB, F), f32)

    vec128 = pl.BlockSpec((F,), lambda i: (0,))

    degp = _make_deg(E, N, NPAD)(col, edge_weight, z1)

    dinv = pl.pallas_call(
        functools.partial(_tc0_body, N),
        out_shape=jax.ShapeDtypeStruct((N, 1), f32),
    )(degp[:, :N])

    tc1 = pl.pallas_call(
        _tc1_body,
        grid=(G,),
        in_specs=[_row_spec(F),
                  pl.BlockSpec((F, F), lambda i: (0, 0)),
                  _row_spec(1)],
        out_specs=_row_spec(F),
        out_shape=jax.ShapeDtypeStruct((N, F), f32),
    )
    hp = tc1(x, W1, dinv)

    prop = _make_prop(E, N, NPAD)

    def tca(relu, s, hp_, b):
        return pl.pallas_call(
            functools.partial(_tca_body, relu),
            grid=(G,),
            in_specs=[pl.BlockSpec((2, RB, F), lambda i: (0, i, 0)),
                      _row_spec(F), _row_spec(1), vec128],
            out_specs=(_row_spec(F),
                       pl.BlockSpec((2, F), lambda i: (0, 0))),
            out_shape=(jax.ShapeDtypeStruct((N, F), f32),
                       jax.ShapeDtypeStruct((2, F), f32)),
        )(s, hp_, dinv, b)

    def tcb(r, st, g, be, wn):
        return pl.pallas_call(
            functools.partial(_tcb_body, N),
            grid=(G,),
            in_specs=[_row_spec(F),
                      pl.BlockSpec((2, F), lambda i: (0, 0)),
                      vec128, vec128,
                      pl.BlockSpec((F, F), lambda i: (0, 0)),
                      _row_spec(1)],
            out_specs=(_row_spec(F), _row_spec(F)),
            out_shape=(jax.ShapeDtypeStruct((N, F), f32),
                       jax.ShapeDtypeStruct((N, F), f32)),
        )(r, st, g, be, wn, dinv)

    s1 = prop(hp, row, col, edge_weight, z2)
    r1, st1 = tca(True, s1, hp, b1)
    x1, hp2 = tcb(r1, st1, g1, be1, W2)
    s2 = prop(hp2, row, col, edge_weight, z2)
    r2, st2 = tca(True, s2, hp2, b2)
    x2, hp3 = tcb(r2, st2, g2, be2, W3)
    s3 = prop(hp3, row, col, edge_weight, z2)
    r3, st3 = tca(False, s3, hp3, b3)

    logits, probs, embed = pl.pallas_call(
        functools.partial(_tc3_body, N),
        grid=(G,),
        in_specs=[_row_spec(F),
                  pl.BlockSpec((2, F), lambda i: (0, 0)),
                  vec128, vec128,
                  _row_spec(F), _row_spec(F),
                  pl.BlockSpec((3 * F, nclass), lambda i: (0, 0)),
                  pl.BlockSpec((nclass,), lambda i: (0,))],
        out_specs=(_row_spec(nclass), _row_spec(nclass), _row_spec(3 * F)),
        out_shape=(jax.ShapeDtypeStruct((N, nclass), f32),
                   jax.ShapeDtypeStruct((N, nclass), f32),
                   jax.ShapeDtypeStruct((N, 3 * F), f32)),
    )(r3, st3, g3, be3, x1, x2, linW, linb)
    return (logits, probs, embed)


# speedup vs baseline: 21.3263x; 1.5557x over previous
import functools
import jax
import jax.numpy as jnp
from jax import lax
from jax.experimental import pallas as pl
from jax.experimental.pallas import tpu as pltpu
from jax.experimental.pallas import tpu_sc as plsc

F = 128
EPSBN = 1e-5
NWORKERS = 32
CH = 128
TPB = 640


def _mesh():
    return plsc.VectorSubcoreMesh(core_axis_name="c", subcore_axis_name="s")


def _make_deg(E, N, NPAD):
    epw = E // NWORKERS
    nfull = epw // CH
    tail = epw - nfull * CH

    @functools.partial(
        pl.kernel,
        out_type=jax.ShapeDtypeStruct((2, NPAD), jnp.float32),
        mesh=_mesh(),
        compiler_params=pltpu.CompilerParams(needs_layout_passes=False),
        scratch_types=[
            pltpu.VMEM((epw,), jnp.int32),
            pltpu.VMEM((epw,), jnp.float32),
            pltpu.VMEM((2, CH), jnp.int32),
            pltpu.VMEM((16,), jnp.int32),
            pltpu.VMEM((TPB,), jnp.float32),
            pltpu.VMEM_SHARED((NPAD,), jnp.float32),
            pltpu.SemaphoreType.DMA,
            pltpu.SemaphoreType.DMA((2,)),
        ],
    )
    def deg_kernel(col_hbm, ew_hbm, out_hbm, col_all, ew_all, colsc_v, col_t,
                   zb_v, acc_sh_deg, sem_p, sem_s):
        cid = lax.axis_index("c")
        sid = lax.axis_index("s")
        wid = sid * 2 + cid
        base = wid * epw
        pltpu.async_copy(col_hbm.at[pl.ds(base, epw)], col_all, sem_p)
        pltpu.async_copy(ew_hbm.at[pl.ds(base, epw)], ew_all, sem_p)
        for jj in range(TPB // 16):
            zb_v[pl.ds(jj * 16, 16)] = jnp.zeros((16,), jnp.float32)
        pltpu.sync_copy(zb_v, acc_sh_deg.at[pl.ds(sid * TPB, TPB)])
        pltpu.make_async_copy(col_hbm.at[pl.ds(0, epw)], col_all,
                              sem_p).wait()
        pltpu.make_async_copy(ew_hbm.at[pl.ds(0, epw)], ew_all, sem_p).wait()
        plsc.subcore_barrier()

        def load_colsc(j, b):
            for jj in range(CH // 16):
                sl = pl.ds(jj * 16, 16)
                colsc_v[b, sl] = col_all[pl.ds(j * CH + jj * 16, 16)]

        def issue_scatter(j, b):
            pltpu.async_copy(ew_all.at[pl.ds(j * CH, CH)],
                             acc_sh_deg.at[colsc_v.at[b]], sem_s.at[b],
                             add=True)

        def wait_scatter(j, b):
            pltpu.make_async_copy(ew_all.at[pl.ds(0, CH)],
                                  acc_sh_deg.at[colsc_v.at[b]],
                                  sem_s.at[b]).wait()

        load_colsc(0, 0)
        issue_scatter(0, 0)
        load_colsc(1, 1)
        issue_scatter(1, 1)

        def body(j, _):
            b = lax.rem(j, 2)
            wait_scatter(j - 2, b)
            load_colsc(j, b)
            issue_scatter(j, b)
            return 0

        lax.fori_loop(2, nfull, body, 0)
        wait_scatter(nfull - 2, nfull % 2)
        wait_scatter(nfull - 1, (nfull - 1) % 2)
        if tail:
            off = nfull * CH
            col_t[...] = col_all[pl.ds(off, tail)]
            pltpu.sync_copy(ew_all.at[pl.ds(off, tail)],
                            acc_sh_deg.at[col_t], add=True)
        plsc.subcore_barrier()
        pltpu.sync_copy(acc_sh_deg.at[pl.ds(sid * TPB, TPB)],
                        out_hbm.at[cid, pl.ds(sid * TPB, TPB)])

    return deg_kernel


def _make_prop(E, N, NPAD):
    epw = E // NWORKERS
    nfull = epw // CH
    tail = epw - nfull * CH
    nhalf = nfull // 2
    assert nfull % 2 == 0

    @functools.partial(
        pl.kernel,
        out_type=jax.ShapeDtypeStruct((2, NPAD, F), jnp.float32),
        mesh=_mesh(),
        compiler_params=pltpu.CompilerParams(needs_layout_passes=False),
        scratch_types=[
            pltpu.VMEM((epw,), jnp.float32),
            pltpu.VMEM((2, CH), jnp.int32),
            pltpu.VMEM((2, CH), jnp.int32),
            pltpu.VMEM((2, CH), jnp.int32),
            pltpu.VMEM((2, CH, F), jnp.float32),
            pltpu.VMEM((16,), jnp.int32),
            pltpu.VMEM((16,), jnp.int32),
            pltpu.VMEM((16, F), jnp.float32),
            pltpu.VMEM_SHARED((NPAD, F), jnp.float32),
            pltpu.SemaphoreType.DMA,
            pltpu.SemaphoreType.DMA((2,)),
            pltpu.SemaphoreType.DMA((2,)),
            pltpu.SemaphoreType.DMA((2,)),
        ],
    )
    def prop_kernel(hp_hbm, row_hbm, col_hbm, ew_hbm, z_hbm, out_hbm,
                    ew_all, rowi_v, coli_v, colsc_v, rows_v,
                    row_t, col_t, rows_t,
                    acc_sh, sem_p, sem_i, sem_g, sem_s):
        cid = lax.axis_index("c")
        sid = lax.axis_index("s")
        wid = sid * 2 + cid
        base = wid * epw
        pltpu.async_copy(ew_hbm.at[pl.ds(base, epw)], ew_all, sem_p)
        for k in range(TPB // CH):
            pltpu.sync_copy(z_hbm, acc_sh.at[pl.ds(sid * TPB + k * CH, CH)])
        pltpu.make_async_copy(ew_hbm.at[pl.ds(0, epw)], ew_all, sem_p).wait()
        plsc.subcore_barrier()

        def issue_idx(j, b):
            off = base + j * CH
            pltpu.async_copy(row_hbm.at[pl.ds(off, CH)], rowi_v.at[b],
                             sem_i.at[b])
            pltpu.async_copy(col_hbm.at[pl.ds(off, CH)], coli_v.at[b],
                             sem_i.at[b])

        def wait_idx(b):
            pltpu.make_async_copy(row_hbm.at[pl.ds(0, CH)], rowi_v.at[b],
                                  sem_i.at[b]).wait()
            pltpu.make_async_copy(col_hbm.at[pl.ds(0, CH)], coli_v.at[b],
                                  sem_i.at[b]).wait()

        def issue_gather(b):
            pltpu.async_copy(hp_hbm.at[rowi_v.at[b]], rows_v.at[b],
                             sem_g.at[b])

        def wait_gather(b):
            pltpu.make_async_copy(hp_hbm.at[rowi_v.at[b]], rows_v.at[b],
                                  sem_g.at[b]).wait()

        def issue_scatter(b):
            pltpu.async_copy(rows_v.at[b], acc_sh.at[colsc_v.at[b]],
                             sem_s.at[b], add=True)

        def wait_scatter(b):
            pltpu.make_async_copy(rows_v.at[b], acc_sh.at[colsc_v.at[b]],
                                  sem_s.at[b]).wait()

        def copy_colsc(b):
            for jj in range(CH // 16):
                sl = pl.ds(jj * 16, 16)
                colsc_v[b, sl] = coli_v[b, sl]

        def scale(j, b):
            ebase = j * CH

            @plsc.parallel_loop(0, CH, unroll=4)
            def _(e):
                ewb = plsc.load_gather(
                    ew_all, [jnp.full((16,), ebase + e, dtype=jnp.int32)])
                for jj in range(F // 16):
                    sl = pl.ds(jj * 16, 16)
                    rows_v[b, e, sl] = rows_v[b, e, sl] * ewb

        issue_idx(0, 0)
        issue_idx(1, 1)
        wait_idx(0)
        issue_gather(0)

        def body(t, _):
            j0 = 2 * t
            wait_gather(0)
            copy_colsc(0)

            @pl.when(t < nhalf - 1)
            def _():
                issue_idx(j0 + 2, 0)

            wait_idx(1)

            @pl.when(t > 0)
            def _():
                wait_scatter(1)

            issue_gather(1)
            scale(j0, 0)
            issue_scatter(0)
            wait_gather(1)
            copy_colsc(1)

            @pl.when(t < nhalf - 1)
            def _():
                issue_idx(j0 + 3, 1)
                wait_idx(0)
                wait_scatter(0)
                issue_gather(0)

            scale(j0 + 1, 1)
            issue_scatter(1)
            return 0

        lax.fori_loop(0, nhalf, body, 0)
        wait_scatter(0)
        wait_scatter(1)
        if tail:
            off = nfull * CH
            pltpu.sync_copy(row_hbm.at[pl.ds(base + off, tail)], row_t)
            pltpu.sync_copy(col_hbm.at[pl.ds(base + off, tail)], col_t)
            pltpu.async_copy(hp_hbm.at[row_t], rows_t, sem_p).wait()

            def sbody(e, _):
                ewb = plsc.load_gather(
                    ew_all, [jnp.full((16,), off + e, dtype=jnp.int32)])
                for jj in range(F // 16):
                    sl = pl.ds(jj * 16, 16)
                    rows_t[e, sl] = rows_t[e, sl] * ewb
                return 0

            lax.fori_loop(0, tail, sbody, 0)
            pltpu.sync_copy(rows_t, acc_sh.at[col_t], add=True)
        plsc.subcore_barrier()
        pltpu.sync_copy(acc_sh.at[pl.ds(sid * TPB, TPB)],
                        out_hbm.at[cid, pl.ds(sid * TPB, TPB)])

    return prop_kernel


RB = 2000


def _mm(a, b):
    return jax.lax.dot_general(a, b, (((1,), (0,)), ((), ())),
                               precision=jax.lax.Precision.HIGHEST,
                               preferred_element_type=jnp.float32)


def _tc0_body(N, degp_ref, dinv_ref):
    deg = degp_ref[0, :] + degp_ref[1, :] + 1.0
    dinv_ref[...] = jax.lax.rsqrt(deg)[:, None]


def _tc1_body(x_ref, w_ref, dinv_ref, hp_ref):
    hp_ref[...] = _mm(x_ref[...], w_ref[...]) * dinv_ref[...]


def _tca_body(relu, s_ref, hp_ref, dinv_ref, b_ref, r_ref, st_ref):
    i = pl.program_id(0)
    t = (s_ref[0] + s_ref[1] + hp_ref[...]) * dinv_ref[...] + b_ref[...]
    if relu:
        t = jnp.maximum(t, 0.0)
    r_ref[...] = t

    @pl.when(i == 0)
    def _():
        st_ref[...] = jnp.zeros_like(st_ref)

    st_ref[0, :] += jnp.sum(t, axis=0)
    st_ref[1, :] += jnp.sum(t * t, axis=0)


def _stats(N, st_ref):
    mean = st_ref[0, :] * (1.0 / N)
    var = st_ref[1, :] * (1.0 / N) - mean * mean
    return mean[None, :], jax.lax.rsqrt(var + EPSBN)[None, :]


def _tcb_body(N, r_ref, st_ref, g_ref, be_ref, wn_ref, dinv_ref,
              x_ref, hn_ref):
    mean, rstd = _stats(N, st_ref)
    xn = (r_ref[...] - mean) * rstd * g_ref[...] + be_ref[...]
    x_ref[...] = xn
    hn_ref[...] = _mm(xn, wn_ref[...]) * dinv_ref[...]


def _tc3_body(N, r_ref, st_ref, g_ref, be_ref, x1_ref, x2_ref,
              linw_ref, linb_ref, logits_ref, probs_ref, embed_ref):
    mean, rstd = _stats(N, st_ref)
    x3 = (r_ref[...] - mean) * rstd * g_ref[...] + be_ref[...]
    x1 = x1_ref[...]
    x2 = x2_ref[...]
    embed_ref[:, 0:F] = x1
    embed_ref[:, F:2 * F] = x2
    embed_ref[:, 2 * F:3 * F] = x3
    logits = (_mm(x1, linw_ref[0:F, :]) + _mm(x2, linw_ref[F:2 * F, :])
              + _mm(x3, linw_ref[2 * F:3 * F, :]) + linb_ref[...])
    logits_ref[...] = logits
    m = jnp.max(logits, axis=1, keepdims=True)
    e = jnp.exp(logits - m)
    probs_ref[...] = e / jnp.sum(e, axis=1, keepdims=True)


def _row_spec(width):
    return pl.BlockSpec((RB, width), lambda i: (i, 0))


def _full_spec():
    return pl.BlockSpec(lambda i: tuple())


def kernel(x, edge_index, edge_weight, W1, b1, W2, b2, W3, b3,
           g1, be1, g2, be2, g3, be3, linW, linb):
    N = x.shape[0]
    E = edge_weight.shape[0]
    NPAD = 16 * TPB
    G = N // RB
    nclass = linW.shape[1]
    f32 = jnp.float32
    row = edge_index[0]
    col = edge_index[1]
    z2 = jnp.zeros((CH, F), f32)

    vec128 = pl.BlockSpec((F,), lambda i: (0,))

    degp = _make_deg(E, N, NPAD)(col, edge_weight)

    dinv = pl.pallas_call(
        functools.partial(_tc0_body, N),
        out_shape=jax.ShapeDtypeStruct((N, 1), f32),
    )(degp[:, :N])

    tc1 = pl.pallas_call(
        _tc1_body,
        grid=(G,),
        in_specs=[_row_spec(F),
                  pl.BlockSpec((F, F), lambda i: (0, 0)),
                  _row_spec(1)],
        out_specs=_row_spec(F),
        out_shape=jax.ShapeDtypeStruct((N, F), f32),
    )
    hp = tc1(x, W1, dinv)

    prop = _make_prop(E, N, NPAD)

    def tca(relu, s, hp_, b):
        return pl.pallas_call(
            functools.partial(_tca_body, relu),
            grid=(G,),
            in_specs=[pl.BlockSpec((2, RB, F), lambda i: (0, i, 0)),
                      _row_spec(F), _row_spec(1), vec128],
            out_specs=(_row_spec(F),
                       pl.BlockSpec((2, F), lambda i: (0, 0))),
            out_shape=(jax.ShapeDtypeStruct((N, F), f32),
                       jax.ShapeDtypeStruct((2, F), f32)),
        )(s, hp_, dinv, b)

    def tcb(r, st, g, be, wn):
        return pl.pallas_call(
            functools.partial(_tcb_body, N),
            grid=(G,),
            in_specs=[_row_spec(F),
                      pl.BlockSpec((2, F), lambda i: (0, 0)),
                      vec128, vec128,
                      pl.BlockSpec((F, F), lambda i: (0, 0)),
                      _row_spec(1)],
            out_specs=(_row_spec(F), _row_spec(F)),
            out_shape=(jax.ShapeDtypeStruct((N, F), f32),
                       jax.ShapeDtypeStruct((N, F), f32)),
        )(r, st, g, be, wn, dinv)

    s1 = prop(hp, row, col, edge_weight, z2)
    r1, st1 = tca(True, s1, hp, b1)
    x1, hp2 = tcb(r1, st1, g1, be1, W2)
    s2 = prop(hp2, row, col, edge_weight, z2)
    r2, st2 = tca(True, s2, hp2, b2)
    x2, hp3 = tcb(r2, st2, g2, be2, W3)
    s3 = prop(hp3, row, col, edge_weight, z2)
    r3, st3 = tca(False, s3, hp3, b3)

    logits, probs, embed = pl.pallas_call(
        functools.partial(_tc3_body, N),
        grid=(G,),
        in_specs=[_row_spec(F),
                  pl.BlockSpec((2, F), lambda i: (0, 0)),
                  vec128, vec128,
                  _row_spec(F), _row_spec(F),
                  pl.BlockSpec((3 * F, nclass), lambda i: (0, 0)),
                  pl.BlockSpec((nclass,), lambda i: (0,))],
        out_specs=(_row_spec(nclass), _row_spec(nclass), _row_spec(3 * F)),
        out_shape=(jax.ShapeDtypeStruct((N, nclass), f32),
                   jax.ShapeDtypeStruct((N, nclass), f32),
                   jax.ShapeDtypeStruct((N, 3 * F), f32)),
    )(r3, st3, g3, be3, x1, x2, linW, linb)
    return (logits, probs, embed)
